# Initial kernel scaffold; baseline (speedup 1.0000x reference)
import jax
import jax.numpy as jnp
from jax.experimental import pallas as pl


def kernel(pred_logits, target):
    raise NotImplementedError("write your pallas kernel here")


# speedup vs baseline: 47.1379x; 47.1379x over previous
import functools

import jax
import jax.numpy as jnp
from jax.experimental import pallas as pl
from jax.experimental.pallas import tpu as pltpu

_MIN_KEPT = 100000
_DICE_W = 0.5
_BCE_W = 0.5
_SMOOTH = 1.0


def _loss_kernel(logits_ref, tgt_ref, out_ref, bits_ref, num_ref, den_ref, *, k, B):
    i = pl.program_id(0)
    l = logits_ref[...]
    t = tgt_ref[...].astype(jnp.float32)
    bce = jnp.maximum(l, 0.0) - l * t + jnp.log1p(jnp.exp(-jnp.abs(l)))
    bits_ref[i] = jax.lax.bitcast_convert_type(bce[0], jnp.int32)
    prob = jax.nn.sigmoid(l)
    num_ref[i] = 2.0 * jnp.sum(prob * t) + _SMOOTH
    den_ref[i] = jnp.sum(prob) + jnp.sum(t) + _SMOOTH

    @pl.when(i == B - 1)
    def _finalize():
        bits = bits_ref[...]

        def body(_, carry):
            lo, hi = carry
            mid = lo + ((hi - lo + 1) >> 1)
            c = jnp.sum((bits >= mid).astype(jnp.int32))
            big = c >= k
            return jnp.where(big, mid, lo), jnp.where(big, hi, mid - 1)

        lo, _ = jax.lax.fori_loop(
            0, 31, body, (jnp.int32(0), jnp.int32(0x7F7FFFFF))
        )
        tau = jax.lax.bitcast_convert_type(lo, jnp.float32)
        gt = bits > lo
        bce_all = jax.lax.bitcast_convert_type(bits, jnp.float32)
        s = jnp.sum(jnp.where(gt, bce_all, 0.0))
        c = jnp.sum(gt.astype(jnp.int32))
        bce_sum = s + (k - c).astype(jnp.float32) * tau
        bce_loss = bce_sum / jnp.float32(k)

        dice_acc = jax.lax.fori_loop(
            0, B, lambda j, a: a + (1.0 - num_ref[j] / den_ref[j]), jnp.float32(0.0)
        )
        dice_loss = dice_acc / jnp.float32(B)
        total = _DICE_W * dice_loss + _BCE_W * bce_loss
        out_ref[...] = jnp.broadcast_to(total, (1, 1))


@jax.jit
def kernel(pred_logits, target):
    B = pred_logits.shape[0]
    N = pred_logits.shape[-1] * pred_logits.shape[-2]
    k = min(_MIN_KEPT * B, B * N)
    M = N // 8
    logits = pred_logits.reshape(B, 8, M)
    tgt = target.reshape(B, 8, M)
    out = pl.pallas_call(
        functools.partial(_loss_kernel, k=k, B=B),
        grid=(B,),
        in_specs=[
            pl.BlockSpec((1, 8, M), lambda i: (i, 0, 0)),
            pl.BlockSpec((1, 8, M), lambda i: (i, 0, 0)),
        ],
        out_specs=pl.BlockSpec((1, 1), lambda i: (0, 0)),
        out_shape=jax.ShapeDtypeStruct((1, 1), jnp.float32),
        scratch_shapes=[
            pltpu.VMEM((B, 8, M), jnp.int32),
            pltpu.SMEM((B,), jnp.float32),
            pltpu.SMEM((B,), jnp.float32),
        ],
    )(logits, tgt)
    return out[0, 0]


# speedup vs baseline: 49.5457x; 1.0511x over previous
import functools

import jax
import jax.numpy as jnp
from jax.experimental import pallas as pl
from jax.experimental.pallas import tpu as pltpu

_MIN_KEPT = 100000
_DICE_W = 0.5
_BCE_W = 0.5
_SMOOTH = 1.0


def _count_ge(arr16, mid):
    m = (arr16 >= mid.astype(jnp.int16)).astype(jnp.int16)
    return jnp.sum(jnp.sum(m, axis=(0, 1)).astype(jnp.int32))


def _loss_kernel(
    logits_ref, tgt_ref, out_ref, bits_ref, hi_ref, lo_ref, cand_ref,
    num_ref, den_ref, *, k, B
):
    i = pl.program_id(0)
    l = logits_ref[...]
    t = tgt_ref[...].astype(jnp.float32)
    bce = jnp.maximum(l, 0.0) - l * t + jnp.log1p(jnp.exp(-jnp.abs(l)))
    bits = jax.lax.bitcast_convert_type(bce, jnp.int32)[0]
    bits_ref[i] = bits
    hi_ref[i] = (bits >> 16).astype(jnp.int16)
    lo_ref[i] = (bits ^ 0x8000).astype(jnp.int16)
    prob = jax.nn.sigmoid(l)
    num_ref[i] = 2.0 * jnp.sum(prob * t) + _SMOOTH
    den_ref[i] = jnp.sum(prob) + jnp.sum(t) + _SMOOTH

    @pl.when(i == B - 1)
    def _finalize():
        hi = hi_ref[...]

        def p1(_, carry):
            lo_, hi_ = carry
            mid = lo_ + ((hi_ - lo_ + 1) >> 1)
            big = _count_ge(hi, mid) >= k
            return jnp.where(big, mid, lo_), jnp.where(big, hi_, mid - 1)

        b, _ = jax.lax.fori_loop(0, 15, p1, (jnp.int32(0), jnp.int32(32767)))
        b16 = b.astype(jnp.int16)

        cand_ref[...] = jnp.where(hi == b16, lo_ref[...], jnp.int16(-32768))
        c_gt = _count_ge(hi, b + 1)

        def p2(_, carry):
            lo_, hi_ = carry
            mid = lo_ + ((hi_ - lo_ + 1) >> 1)
            big = c_gt + _count_ge(cand_ref[...], mid) >= k
            return jnp.where(big, mid, lo_), jnp.where(big, hi_, mid - 1)

        mlo, _ = jax.lax.fori_loop(
            0, 16, p2, (jnp.int32(-32768), jnp.int32(32767))
        )
        tau_bits = (b << 16) | ((mlo ^ 0x8000) & 0xFFFF)
        tau = jax.lax.bitcast_convert_type(tau_bits, jnp.float32)

        bits_all = bits_ref[...]
        gt = bits_all > tau_bits
        bce_all = jax.lax.bitcast_convert_type(bits_all, jnp.float32)
        s = jnp.sum(jnp.where(gt, bce_all, 0.0))
        c = jnp.sum(gt.astype(jnp.int32))
        bce_sum = s + (k - c).astype(jnp.float32) * tau
        bce_loss = bce_sum / jnp.float32(k)

        dice_acc = jax.lax.fori_loop(
            0, B, lambda j, a: a + (1.0 - num_ref[j] / den_ref[j]), jnp.float32(0.0)
        )
        dice_loss = dice_acc / jnp.float32(B)
        total = _DICE_W * dice_loss + _BCE_W * bce_loss
        out_ref[...] = jnp.broadcast_to(total, (1, 1))


@jax.jit
def kernel(pred_logits, target):
    B = pred_logits.shape[0]
    N = pred_logits.shape[-1] * pred_logits.shape[-2]
    k = min(_MIN_KEPT * B, B * N)
    M = N // 16
    logits = pred_logits.reshape(B, 16, M)
    tgt = target.reshape(B, 16, M)
    out = pl.pallas_call(
        functools.partial(_loss_kernel, k=k, B=B),
        grid=(B,),
        in_specs=[
            pl.BlockSpec((1, 16, M), lambda i: (i, 0, 0)),
            pl.BlockSpec((1, 16, M), lambda i: (i, 0, 0)),
        ],
        out_specs=pl.BlockSpec((1, 1), lambda i: (0, 0)),
        out_shape=jax.ShapeDtypeStruct((1, 1), jnp.float32),
        scratch_shapes=[
            pltpu.VMEM((B, 16, M), jnp.int32),
            pltpu.VMEM((B, 16, M), jnp.int16),
            pltpu.VMEM((B, 16, M), jnp.int16),
            pltpu.VMEM((B, 16, M), jnp.int16),
            pltpu.SMEM((B,), jnp.float32),
            pltpu.SMEM((B,), jnp.float32),
        ],
    )(logits, tgt)
    return out[0, 0]


# speedup vs baseline: 68.5669x; 1.3839x over previous
import functools

import jax
import jax.numpy as jnp
from jax.experimental import pallas as pl
from jax.experimental.pallas import tpu as pltpu

_MIN_KEPT = 100000
_DICE_W = 0.5
_BCE_W = 0.5
_SMOOTH = 1.0


def _loss_kernel(logits_ref, tgt_ref, out_ref, bits_ref, num_ref, den_ref, *, k, B):
    i = pl.program_id(0)
    l = logits_ref[...]
    t = tgt_ref[...].astype(jnp.float32)
    bce = jnp.maximum(l, 0.0) - l * t + jnp.log1p(jnp.exp(-jnp.abs(l)))
    bits_ref[i] = jax.lax.bitcast_convert_type(bce[0], jnp.int32)
    prob = jax.nn.sigmoid(l)
    num_ref[i] = 2.0 * jnp.sum(prob * t) + _SMOOTH
    den_ref[i] = jnp.sum(prob) + jnp.sum(t) + _SMOOTH

    @pl.when(i == B - 1)
    def _finalize():
        bits = bits_ref[...]

        def body(_, carry):
            lo, hi = carry
            mid = lo + ((hi - lo + 1) >> 1)
            m = (bits >= mid).astype(jnp.int32)
            c = jnp.sum(jnp.sum(m, axis=(0, 1)))
            big = c >= k
            return jnp.where(big, mid, lo), jnp.where(big, hi, mid - 1)

        lo, _ = jax.lax.fori_loop(
            0, 31, body, (jnp.int32(0), jnp.int32(0x7F7FFFFF))
        )
        tau = jax.lax.bitcast_convert_type(lo, jnp.float32)
        gt = bits > lo
        bce_all = jax.lax.bitcast_convert_type(bits, jnp.float32)
        s = jnp.sum(jnp.sum(jnp.where(gt, bce_all, 0.0), axis=(0, 1)))
        c = jnp.sum(jnp.sum(gt.astype(jnp.int32), axis=(0, 1)))
        bce_sum = s + (k - c).astype(jnp.float32) * tau
        bce_loss = bce_sum / jnp.float32(k)

        dice_acc = jax.lax.fori_loop(
            0, B, lambda j, a: a + (1.0 - num_ref[j] / den_ref[j]), jnp.float32(0.0)
        )
        dice_loss = dice_acc / jnp.float32(B)
        total = _DICE_W * dice_loss + _BCE_W * bce_loss
        out_ref[...] = jnp.broadcast_to(total, (1, 1))


@jax.jit
def kernel(pred_logits, target):
    B = pred_logits.shape[0]
    N = pred_logits.shape[-1] * pred_logits.shape[-2]
    k = min(_MIN_KEPT * B, B * N)
    M = N // 8
    logits = pred_logits.reshape(B, 8, M)
    tgt = target.reshape(B, 8, M)
    out = pl.pallas_call(
        functools.partial(_loss_kernel, k=k, B=B),
        grid=(B,),
        in_specs=[
            pl.BlockSpec((1, 8, M), lambda i: (i, 0, 0)),
            pl.BlockSpec((1, 8, M), lambda i: (i, 0, 0)),
        ],
        out_specs=pl.BlockSpec((1, 1), lambda i: (0, 0)),
        out_shape=jax.ShapeDtypeStruct((1, 1), jnp.float32),
        scratch_shapes=[
            pltpu.VMEM((B, 8, M), jnp.int32),
            pltpu.SMEM((B,), jnp.float32),
            pltpu.SMEM((B,), jnp.float32),
        ],
    )(logits, tgt)
    return out[0, 0]
